# baseline (speedup 1.0000x reference)
import jax
import jax.numpy as jnp
from jax.experimental import pallas as pl
from jax.experimental.pallas import tpu as pltpu

_SIZE = 65536
_DIM = 128
_BATCH = 1024
_NCH = 16
_CH = _SIZE // _NCH


def _body(out_hbm, bank_hbm, oo_hbm, snap_hbm, new_hbm, buf, vin, vout,
          isem, ssem, nsem, osem):
    def in_cp(j):
        return pltpu.make_async_copy(
            bank_hbm.at[:, pl.ds(j * _CH, _CH)], buf.at[j], isem.at[j])

    def snap_cp(j):
        return pltpu.make_async_copy(
            buf.at[j], snap_hbm.at[:, pl.ds(j * _CH, _CH)], ssem.at[j])

    def new_cp(j):
        if j == 0:
            return pltpu.make_async_copy(
                buf.at[0, :, pl.ds(_BATCH, _CH - _BATCH)],
                new_hbm.at[:, pl.ds(_BATCH, _CH - _BATCH)], nsem.at[0])
        return pltpu.make_async_copy(
            buf.at[j], new_hbm.at[:, pl.ds(j * _CH, _CH)], nsem.at[j])

    ocp_in = pltpu.make_async_copy(out_hbm, vin, osem.at[0])
    ocp_in.start()
    for j in range(_NCH):
        in_cp(j).start()
    ocp_in.wait()
    vout[...] = jnp.transpose(vin[...])
    pltpu.make_async_copy(vin, oo_hbm, osem.at[1]).start()
    pltpu.make_async_copy(vout, new_hbm.at[:, pl.ds(0, _BATCH)], osem.at[2]).start()
    for j in range(_NCH):
        in_cp(j).wait()
        snap_cp(j).start()
        new_cp(j).start()
    for j in range(_NCH):
        snap_cp(j).wait()
        new_cp(j).wait()
    pltpu.make_async_copy(vin, oo_hbm, osem.at[1]).wait()
    pltpu.make_async_copy(vout, new_hbm.at[:, pl.ds(0, _BATCH)], osem.at[2]).wait()


def kernel(output, bank):
    out_shapes = (
        jax.ShapeDtypeStruct((_BATCH, _DIM), output.dtype),
        jax.ShapeDtypeStruct((_DIM, _SIZE), bank.dtype),
        jax.ShapeDtypeStruct((_DIM, _SIZE), bank.dtype),
    )
    out, snap, new = pl.pallas_call(
        _body,
        in_specs=[
            pl.BlockSpec(memory_space=pl.ANY),
            pl.BlockSpec(memory_space=pl.ANY),
        ],
        out_specs=[
            pl.BlockSpec(memory_space=pl.ANY),
            pl.BlockSpec(memory_space=pl.ANY),
            pl.BlockSpec(memory_space=pl.ANY),
        ],
        out_shape=out_shapes,
        scratch_shapes=[
            pltpu.VMEM((_NCH, _DIM, _CH), jnp.float32),
            pltpu.VMEM((_BATCH, _DIM), jnp.float32),
            pltpu.VMEM((_DIM, _BATCH), jnp.float32),
            pltpu.SemaphoreType.DMA((_NCH,)),
            pltpu.SemaphoreType.DMA((_NCH,)),
            pltpu.SemaphoreType.DMA((_NCH,)),
            pltpu.SemaphoreType.DMA((3,)),
        ],
    )(output, bank)
    return (out, snap, new)


# speedup vs baseline: 1.0970x; 1.0970x over previous
import jax
import jax.numpy as jnp
from jax.experimental import pallas as pl
from jax.experimental.pallas import tpu as pltpu

_SIZE = 65536
_DIM = 128
_BATCH = 1024
_NCH = 2
_CH = _SIZE // _NCH


def _body(out_hbm, bank_hbm, oo_hbm, snap_hbm, new_hbm, buf, vin, vout,
          isem, ssem, nsem, osem):
    def in_cp(j):
        return pltpu.make_async_copy(
            bank_hbm.at[:, pl.ds(j * _CH, _CH)], buf.at[j], isem.at[j])

    def snap_cp(j):
        return pltpu.make_async_copy(
            buf.at[j], snap_hbm.at[:, pl.ds(j * _CH, _CH)], ssem.at[j])

    def new_cp(j):
        if j == 0:
            return pltpu.make_async_copy(
                buf.at[0, :, pl.ds(_BATCH, _CH - _BATCH)],
                new_hbm.at[:, pl.ds(_BATCH, _CH - _BATCH)], nsem.at[0])
        return pltpu.make_async_copy(
            buf.at[j], new_hbm.at[:, pl.ds(j * _CH, _CH)], nsem.at[j])

    ocp_in = pltpu.make_async_copy(out_hbm, vin, osem.at[0])
    ocp_in.start()
    for j in range(_NCH):
        in_cp(j).start()
    ocp_in.wait()
    vout[...] = jnp.transpose(vin[...])
    pltpu.make_async_copy(vin, oo_hbm, osem.at[1]).start()
    pltpu.make_async_copy(vout, new_hbm.at[:, pl.ds(0, _BATCH)], osem.at[2]).start()
    for j in range(_NCH):
        in_cp(j).wait()
        snap_cp(j).start()
        new_cp(j).start()
    for j in range(_NCH):
        snap_cp(j).wait()
        new_cp(j).wait()
    pltpu.make_async_copy(vin, oo_hbm, osem.at[1]).wait()
    pltpu.make_async_copy(vout, new_hbm.at[:, pl.ds(0, _BATCH)], osem.at[2]).wait()


def kernel(output, bank):
    out_shapes = (
        jax.ShapeDtypeStruct((_BATCH, _DIM), output.dtype),
        jax.ShapeDtypeStruct((_DIM, _SIZE), bank.dtype),
        jax.ShapeDtypeStruct((_DIM, _SIZE), bank.dtype),
    )
    out, snap, new = pl.pallas_call(
        _body,
        in_specs=[
            pl.BlockSpec(memory_space=pl.ANY),
            pl.BlockSpec(memory_space=pl.ANY),
        ],
        out_specs=[
            pl.BlockSpec(memory_space=pl.ANY),
            pl.BlockSpec(memory_space=pl.ANY),
            pl.BlockSpec(memory_space=pl.ANY),
        ],
        out_shape=out_shapes,
        scratch_shapes=[
            pltpu.VMEM((_NCH, _DIM, _CH), jnp.float32),
            pltpu.VMEM((_BATCH, _DIM), jnp.float32),
            pltpu.VMEM((_DIM, _BATCH), jnp.float32),
            pltpu.SemaphoreType.DMA((_NCH,)),
            pltpu.SemaphoreType.DMA((_NCH,)),
            pltpu.SemaphoreType.DMA((_NCH,)),
            pltpu.SemaphoreType.DMA((3,)),
        ],
    )(output, bank)
    return (out, snap, new)
